# baseline (speedup 1.0000x reference)
import functools
import math

import jax
import jax.numpy as jnp
from jax import lax
from jax.experimental import pallas as pl
from jax.experimental.pallas import tpu as pltpu
from jax.experimental.pallas import tpu_sc as plsc

_B, _S, _H = 1, 2048, 1024
_E, _K, _I = 8, 2, 4096
_T = _B * _S

_MT = 256
_NT = 23
_NTM = _NT * _MT
_NTE = 32

_NW = 32
_TPW = _T // _NW
_CHT = 32


def _router_body(x_ref, wr_ref, br_ref,
                 d0_ref, d1_ref, w0_ref, w1_ref, te_ref, aux_ref):
    x = x_ref[...]
    logits = lax.dot_general(
        x.astype(jnp.bfloat16), wr_ref[...].astype(jnp.bfloat16),
        (((1,), (0,)), ((), ())),
        preferred_element_type=jnp.float32)
    logits = logits + br_ref[...]

    ii = lax.broadcasted_iota(jnp.int32, (_T, _E), 1)
    m1 = jnp.max(logits, axis=1, keepdims=True)
    idx1 = jnp.min(jnp.where(logits == m1, ii, _E), axis=1, keepdims=True)
    masked = jnp.where(ii == idx1, -jnp.inf, logits)
    m2 = jnp.max(masked, axis=1, keepdims=True)
    idx2 = jnp.min(jnp.where(masked == m2, ii, _E), axis=1, keepdims=True)

    e2 = jnp.exp(m2 - m1)
    ww1 = 1.0 / (1.0 + e2)
    ww2 = e2 * ww1
    oh1 = (ii == idx1)
    oh2 = (ii == idx2)
    oh = oh1.astype(jnp.float32) + oh2.astype(jnp.float32)
    cnt = jnp.sum(oh, axis=0, keepdims=True)

    z = jnp.exp(logits - m1)
    probs = z / jnp.sum(z, axis=1, keepdims=True)
    imp = jnp.sum(probs, axis=0, keepdims=True)
    impn = imp / jnp.sum(imp)
    loadn = cnt / jnp.sum(cnt)
    aux_ref[...] = (jnp.sum(impn * loadn) * float(_E)).reshape(1, 1)

    r = lax.broadcasted_iota(jnp.int32, (_T, _T), 0)
    c = lax.broadcasted_iota(jnp.int32, (_T, _T), 1)
    tri = (r > c).astype(jnp.bfloat16)
    excl = lax.dot_general(tri, oh.astype(jnp.bfloat16),
                           (((1,), (0,)), ((), ())),
                           preferred_element_type=jnp.float32)

    cap = jnp.ceil(cnt * (1.0 / _MT)) * float(_MT)
    r8 = lax.broadcasted_iota(jnp.int32, (_E, _E), 0)
    c8 = lax.broadcasted_iota(jnp.int32, (_E, _E), 1)
    u8 = (r8 < c8).astype(jnp.bfloat16)
    off = lax.dot_general(cap.astype(jnp.bfloat16), u8,
                          (((1,), (0,)), ((), ())),
                          preferred_element_type=jnp.float32)

    offb = jnp.broadcast_to(off, (_T, _E))
    d0 = jnp.sum(jnp.where(oh1, offb + excl, 0.0), axis=1, keepdims=True)
    d1 = jnp.sum(jnp.where(oh2, offb + excl, 0.0), axis=1, keepdims=True)
    d0_ref[...] = d0.astype(jnp.int32)
    d1_ref[...] = d1.astype(jnp.int32)
    w0_ref[...] = jnp.broadcast_to(ww1, (_T, 16))
    w1_ref[...] = jnp.broadcast_to(ww2, (_T, 16))

    ts = lax.broadcasted_iota(jnp.int32, (_NTE, _E), 0).astype(jnp.float32)
    ts = ts * float(_MT)
    ends = jnp.broadcast_to(off + cap, (_NTE, _E))
    te = jnp.sum((ts >= ends).astype(jnp.int32), axis=1, keepdims=True)
    te_ref[...] = jnp.minimum(te, _E - 1)


def _gelu_exact(v):
    return v * 0.5 * (1.0 + lax.erf(v * (1.0 / math.sqrt(2.0))))


def _seg_changed(te_ref):
    i = pl.program_id(0)
    prev = te_ref[jnp.maximum(i - 1, 0)]
    return (i == 0) | (te_ref[i] != prev)


def _ffn1_body(te_ref, xs_ref, w1_ref, b1_ref, mid_ref, w1b):
    @pl.when(_seg_changed(te_ref))
    def _():
        w1b[...] = w1_ref[0].astype(jnp.bfloat16)

    x = xs_ref[...].astype(jnp.bfloat16)
    mid = lax.dot_general(
        x, w1b[...], (((1,), (0,)), ((), ())),
        preferred_element_type=jnp.float32)
    mid_ref[...] = _gelu_exact(mid + b1_ref[0]).astype(jnp.bfloat16)


def _ffn2_body(te_ref, mid_ref, w2_ref, b2_ref, out_ref, w2b):
    @pl.when(_seg_changed(te_ref))
    def _():
        w2b[...] = w2_ref[0].astype(jnp.bfloat16)

    out_ref[...] = lax.dot_general(
        mid_ref[...], w2b[...], (((1,), (0,)), ((), ())),
        preferred_element_type=jnp.float32) + b2_ref[0]


@functools.cache
def _sc_kernels():
    mesh = plsc.VectorSubcoreMesh(core_axis_name="c", subcore_axis_name="s")

    @functools.partial(
        pl.kernel,
        out_type=jax.ShapeDtypeStruct((_NTM, _H), jnp.float32),
        mesh=mesh,
        scratch_types=[
            pltpu.VMEM((_TPW, _H), jnp.float32),
            pltpu.VMEM((2, _TPW), jnp.int32),
            pltpu.SemaphoreType.DMA,
        ],
    )
    def dispatch(x_hbm, d_hbm, xs_hbm, xv, idxv, sem):
        wid = lax.axis_index("s") * 2 + lax.axis_index("c")
        base = wid * _TPW
        pltpu.sync_copy(x_hbm.at[pl.ds(base, _TPW)], xv)
        pltpu.sync_copy(d_hbm.at[0, pl.ds(base, _TPW)], idxv.at[0])
        pltpu.sync_copy(d_hbm.at[1, pl.ds(base, _TPW)], idxv.at[1])
        c0 = pltpu.async_copy(xv, xs_hbm.at[idxv.at[0]], sem)
        c1 = pltpu.async_copy(xv, xs_hbm.at[idxv.at[1]], sem)
        c0.wait()
        c1.wait()

    @functools.partial(
        pl.kernel,
        out_type=jax.ShapeDtypeStruct((_T, _H), jnp.float32),
        mesh=mesh,
        scratch_types=[
            pltpu.VMEM((_CHT, _H), jnp.float32),
            pltpu.VMEM((_CHT, _H), jnp.float32),
            pltpu.VMEM((2, _CHT), jnp.int32),
            pltpu.VMEM((_CHT, 16), jnp.float32),
            pltpu.VMEM((_CHT, 16), jnp.float32),
            pltpu.SemaphoreType.DMA,
        ],
    )
    def combine(eout_hbm, d_hbm, w0_hbm, w1_hbm, y_hbm,
                g0, g1, idxv, wv0, wv1, sem):
        _combine_body(eout_hbm, d_hbm, w0_hbm, w1_hbm, y_hbm,
                      g0, g1, idxv, wv0, wv1, sem)

    return dispatch, combine


def _combine_body(eout_hbm, d_hbm, w0_hbm, w1_hbm, y_hbm,
                  g0, g1, idxv, wv0, wv1, sem):
    wid = lax.axis_index("s") * 2 + lax.axis_index("c")
    for ci in range(_TPW // _CHT):
        base = wid * _TPW + ci * _CHT
        pltpu.sync_copy(d_hbm.at[0, pl.ds(base, _CHT)], idxv.at[0])
        pltpu.sync_copy(d_hbm.at[1, pl.ds(base, _CHT)], idxv.at[1])
        pltpu.sync_copy(w0_hbm.at[pl.ds(base, _CHT)], wv0)
        pltpu.sync_copy(w1_hbm.at[pl.ds(base, _CHT)], wv1)
        c0 = pltpu.async_copy(eout_hbm.at[idxv.at[0]], g0, sem)
        c1 = pltpu.async_copy(eout_hbm.at[idxv.at[1]], g1, sem)
        c0.wait()
        c1.wait()

        def tok_body(j, carry):
            a0 = wv0[j, :]
            a1 = wv1[j, :]

            def h_body(h, carry2):
                for u in range(4):
                    s = pl.ds(h * 64 + u * 16, 16)
                    g0[j, s] = g0[j, s] * a0 + g1[j, s] * a1
                return carry2

            return lax.fori_loop(0, _H // 64, h_body, carry)

        lax.fori_loop(0, _CHT, tok_body, 0)
        pltpu.sync_copy(g0, y_hbm.at[pl.ds(base, _CHT)])


def kernel(hidden_states, Wr, br, W1, b1, W2, b2):
    x = hidden_states.reshape(_T, _H)

    d0, d1, w0r, w1r, te, aux = pl.pallas_call(
        _router_body,
        out_shape=(
            jax.ShapeDtypeStruct((_T, 1), jnp.int32),
            jax.ShapeDtypeStruct((_T, 1), jnp.int32),
            jax.ShapeDtypeStruct((_T, 16), jnp.float32),
            jax.ShapeDtypeStruct((_T, 16), jnp.float32),
            jax.ShapeDtypeStruct((_NTE, 1), jnp.int32),
            jax.ShapeDtypeStruct((1, 1), jnp.float32),
        ),
    )(x, Wr, br.reshape(1, _E))

    d = jnp.concatenate([d0.reshape(1, _T), d1.reshape(1, _T)], axis=0)

    _dispatch, _combine = _sc_kernels()
    xs = _dispatch(x, d)

    tef = te.reshape(_NTE)
    mid = pl.pallas_call(
        _ffn1_body,
        grid_spec=pltpu.PrefetchScalarGridSpec(
            num_scalar_prefetch=1,
            grid=(_NT,),
            in_specs=[
                pl.BlockSpec((_MT, _H), lambda i, s: (i, 0)),
                pl.BlockSpec((1, _H, _I), lambda i, s: (s[i], 0, 0)),
                pl.BlockSpec((1, 1, _I), lambda i, s: (s[i], 0, 0)),
            ],
            out_specs=pl.BlockSpec((_MT, _I), lambda i, s: (i, 0)),
            scratch_shapes=[pltpu.VMEM((_H, _I), jnp.bfloat16)],
        ),
        out_shape=jax.ShapeDtypeStruct((_NTM, _I), jnp.bfloat16),
    )(tef, xs, W1, b1.reshape(_E, 1, _I))

    eout = pl.pallas_call(
        _ffn2_body,
        grid_spec=pltpu.PrefetchScalarGridSpec(
            num_scalar_prefetch=1,
            grid=(_NT,),
            in_specs=[
                pl.BlockSpec((_MT, _I), lambda i, s: (i, 0)),
                pl.BlockSpec((1, _I, _H), lambda i, s: (s[i], 0, 0)),
                pl.BlockSpec((1, 1, _H), lambda i, s: (s[i], 0, 0)),
            ],
            out_specs=pl.BlockSpec((_MT, _H), lambda i, s: (i, 0)),
            scratch_shapes=[pltpu.VMEM((_I, _H), jnp.bfloat16)],
        ),
        out_shape=jax.ShapeDtypeStruct((_NTM, _H), jnp.float32),
    )(tef, mid, W2, b2.reshape(_E, 1, _H))

    y = _combine(eout, d, w0r, w1r)

    return y.reshape(_B, _S, _H), aux.reshape(())


# speedup vs baseline: 1.0068x; 1.0068x over previous
import functools
import math

import jax
import jax.numpy as jnp
from jax import lax
from jax.experimental import pallas as pl
from jax.experimental.pallas import tpu as pltpu
from jax.experimental.pallas import tpu_sc as plsc

_B, _S, _H = 1, 2048, 1024
_E, _K, _I = 8, 2, 4096
_T = _B * _S

_MT = 128
_NT = 39
_NTM = _NT * _MT
_NTE = 40

_NW = 32
_TPW = _T // _NW
_CHT = 32


def _router_body(x_ref, wr_ref, br_ref,
                 d0_ref, d1_ref, w0_ref, w1_ref, te_ref, aux_ref):
    x = x_ref[...]
    logits = lax.dot_general(
        x.astype(jnp.bfloat16), wr_ref[...].astype(jnp.bfloat16),
        (((1,), (0,)), ((), ())),
        preferred_element_type=jnp.float32)
    logits = logits + br_ref[...]

    ii = lax.broadcasted_iota(jnp.int32, (_T, _E), 1)
    m1 = jnp.max(logits, axis=1, keepdims=True)
    idx1 = jnp.min(jnp.where(logits == m1, ii, _E), axis=1, keepdims=True)
    masked = jnp.where(ii == idx1, -jnp.inf, logits)
    m2 = jnp.max(masked, axis=1, keepdims=True)
    idx2 = jnp.min(jnp.where(masked == m2, ii, _E), axis=1, keepdims=True)

    e2 = jnp.exp(m2 - m1)
    ww1 = 1.0 / (1.0 + e2)
    ww2 = e2 * ww1
    oh1 = (ii == idx1)
    oh2 = (ii == idx2)
    oh = oh1.astype(jnp.float32) + oh2.astype(jnp.float32)
    cnt = jnp.sum(oh, axis=0, keepdims=True)

    z = jnp.exp(logits - m1)
    probs = z / jnp.sum(z, axis=1, keepdims=True)
    imp = jnp.sum(probs, axis=0, keepdims=True)
    impn = imp / jnp.sum(imp)
    loadn = cnt / jnp.sum(cnt)
    aux_ref[...] = (jnp.sum(impn * loadn) * float(_E)).reshape(1, 1)

    r = lax.broadcasted_iota(jnp.int32, (_T, _T), 0)
    c = lax.broadcasted_iota(jnp.int32, (_T, _T), 1)
    tri = (r > c).astype(jnp.bfloat16)
    excl = lax.dot_general(tri, oh.astype(jnp.bfloat16),
                           (((1,), (0,)), ((), ())),
                           preferred_element_type=jnp.float32)

    cap = jnp.ceil(cnt * (1.0 / _MT)) * float(_MT)
    r8 = lax.broadcasted_iota(jnp.int32, (_E, _E), 0)
    c8 = lax.broadcasted_iota(jnp.int32, (_E, _E), 1)
    u8 = (r8 < c8).astype(jnp.bfloat16)
    off = lax.dot_general(cap.astype(jnp.bfloat16), u8,
                          (((1,), (0,)), ((), ())),
                          preferred_element_type=jnp.float32)

    offb = jnp.broadcast_to(off, (_T, _E))
    d0 = jnp.sum(jnp.where(oh1, offb + excl, 0.0), axis=1, keepdims=True)
    d1 = jnp.sum(jnp.where(oh2, offb + excl, 0.0), axis=1, keepdims=True)
    d0_ref[...] = d0.astype(jnp.int32)
    d1_ref[...] = d1.astype(jnp.int32)
    w0_ref[...] = jnp.broadcast_to(ww1, (_T, 16))
    w1_ref[...] = jnp.broadcast_to(ww2, (_T, 16))

    ts = lax.broadcasted_iota(jnp.int32, (_NTE, _E), 0).astype(jnp.float32)
    ts = ts * float(_MT)
    ends = jnp.broadcast_to(off + cap, (_NTE, _E))
    te = jnp.sum((ts >= ends).astype(jnp.int32), axis=1, keepdims=True)
    te_ref[...] = jnp.minimum(te, _E - 1)


def _gelu_exact(v):
    return v * 0.5 * (1.0 + lax.erf(v * (1.0 / math.sqrt(2.0))))


def _seg_changed(te_ref):
    i = pl.program_id(0)
    prev = te_ref[jnp.maximum(i - 1, 0)]
    return (i == 0) | (te_ref[i] != prev)


def _ffn1_body(te_ref, xs_ref, w1_ref, b1_ref, mid_ref, w1b):
    @pl.when(_seg_changed(te_ref))
    def _():
        w1b[...] = w1_ref[0].astype(jnp.bfloat16)

    x = xs_ref[...].astype(jnp.bfloat16)
    mid = lax.dot_general(
        x, w1b[...], (((1,), (0,)), ((), ())),
        preferred_element_type=jnp.float32)
    mid_ref[...] = _gelu_exact(mid + b1_ref[0]).astype(jnp.bfloat16)


def _ffn2_body(te_ref, mid_ref, w2_ref, b2_ref, out_ref, w2b):
    @pl.when(_seg_changed(te_ref))
    def _():
        w2b[...] = w2_ref[0].astype(jnp.bfloat16)

    out_ref[...] = lax.dot_general(
        mid_ref[...], w2b[...], (((1,), (0,)), ((), ())),
        preferred_element_type=jnp.float32) + b2_ref[0]


@functools.cache
def _sc_kernels():
    mesh = plsc.VectorSubcoreMesh(core_axis_name="c", subcore_axis_name="s")

    @functools.partial(
        pl.kernel,
        out_type=jax.ShapeDtypeStruct((_NTM, _H), jnp.float32),
        mesh=mesh,
        scratch_types=[
            pltpu.VMEM((_TPW, _H), jnp.float32),
            pltpu.VMEM((2, _TPW), jnp.int32),
            pltpu.SemaphoreType.DMA,
        ],
    )
    def dispatch(x_hbm, d_hbm, xs_hbm, xv, idxv, sem):
        wid = lax.axis_index("s") * 2 + lax.axis_index("c")
        base = wid * _TPW
        pltpu.sync_copy(x_hbm.at[pl.ds(base, _TPW)], xv)
        pltpu.sync_copy(d_hbm.at[0, pl.ds(base, _TPW)], idxv.at[0])
        pltpu.sync_copy(d_hbm.at[1, pl.ds(base, _TPW)], idxv.at[1])
        c0 = pltpu.async_copy(xv, xs_hbm.at[idxv.at[0]], sem)
        c1 = pltpu.async_copy(xv, xs_hbm.at[idxv.at[1]], sem)
        c0.wait()
        c1.wait()

    @functools.partial(
        pl.kernel,
        out_type=jax.ShapeDtypeStruct((_T, _H), jnp.float32),
        mesh=mesh,
        scratch_types=[
            pltpu.VMEM((_CHT, _H), jnp.float32),
            pltpu.VMEM((_CHT, _H), jnp.float32),
            pltpu.VMEM((2, _CHT), jnp.int32),
            pltpu.VMEM((_CHT, 16), jnp.float32),
            pltpu.VMEM((_CHT, 16), jnp.float32),
            pltpu.SemaphoreType.DMA,
        ],
    )
    def combine(eout_hbm, d_hbm, w0_hbm, w1_hbm, y_hbm,
                g0, g1, idxv, wv0, wv1, sem):
        _combine_body(eout_hbm, d_hbm, w0_hbm, w1_hbm, y_hbm,
                      g0, g1, idxv, wv0, wv1, sem)

    return dispatch, combine


def _combine_body(eout_hbm, d_hbm, w0_hbm, w1_hbm, y_hbm,
                  g0, g1, idxv, wv0, wv1, sem):
    wid = lax.axis_index("s") * 2 + lax.axis_index("c")
    for ci in range(_TPW // _CHT):
        base = wid * _TPW + ci * _CHT
        pltpu.sync_copy(d_hbm.at[0, pl.ds(base, _CHT)], idxv.at[0])
        pltpu.sync_copy(d_hbm.at[1, pl.ds(base, _CHT)], idxv.at[1])
        pltpu.sync_copy(w0_hbm.at[pl.ds(base, _CHT)], wv0)
        pltpu.sync_copy(w1_hbm.at[pl.ds(base, _CHT)], wv1)
        c0 = pltpu.async_copy(eout_hbm.at[idxv.at[0]], g0, sem)
        c1 = pltpu.async_copy(eout_hbm.at[idxv.at[1]], g1, sem)
        c0.wait()
        c1.wait()

        def tok_body(j, carry):
            a0 = wv0[j, :]
            a1 = wv1[j, :]

            def h_body(h, carry2):
                for u in range(4):
                    s = pl.ds(h * 64 + u * 16, 16)
                    g0[j, s] = g0[j, s] * a0 + g1[j, s] * a1
                return carry2

            return lax.fori_loop(0, _H // 64, h_body, carry)

        lax.fori_loop(0, _CHT, tok_body, 0)
        pltpu.sync_copy(g0, y_hbm.at[pl.ds(base, _CHT)])


def kernel(hidden_states, Wr, br, W1, b1, W2, b2):
    x = hidden_states.reshape(_T, _H)

    d0, d1, w0r, w1r, te, aux = pl.pallas_call(
        _router_body,
        out_shape=(
            jax.ShapeDtypeStruct((_T, 1), jnp.int32),
            jax.ShapeDtypeStruct((_T, 1), jnp.int32),
            jax.ShapeDtypeStruct((_T, 16), jnp.float32),
            jax.ShapeDtypeStruct((_T, 16), jnp.float32),
            jax.ShapeDtypeStruct((_NTE, 1), jnp.int32),
            jax.ShapeDtypeStruct((1, 1), jnp.float32),
        ),
    )(x, Wr, br.reshape(1, _E))

    d = jnp.concatenate([d0.reshape(1, _T), d1.reshape(1, _T)], axis=0)

    _dispatch, _combine = _sc_kernels()
    xs = _dispatch(x, d)

    tef = te.reshape(_NTE)
    mid = pl.pallas_call(
        _ffn1_body,
        grid_spec=pltpu.PrefetchScalarGridSpec(
            num_scalar_prefetch=1,
            grid=(_NT,),
            in_specs=[
                pl.BlockSpec((_MT, _H), lambda i, s: (i, 0)),
                pl.BlockSpec((1, _H, _I), lambda i, s: (s[i], 0, 0)),
                pl.BlockSpec((1, 1, _I), lambda i, s: (s[i], 0, 0)),
            ],
            out_specs=pl.BlockSpec((_MT, _I), lambda i, s: (i, 0)),
            scratch_shapes=[pltpu.VMEM((_H, _I), jnp.bfloat16)],
        ),
        out_shape=jax.ShapeDtypeStruct((_NTM, _I), jnp.bfloat16),
    )(tef, xs, W1, b1.reshape(_E, 1, _I))

    eout = pl.pallas_call(
        _ffn2_body,
        grid_spec=pltpu.PrefetchScalarGridSpec(
            num_scalar_prefetch=1,
            grid=(_NT,),
            in_specs=[
                pl.BlockSpec((_MT, _I), lambda i, s: (i, 0)),
                pl.BlockSpec((1, _I, _H), lambda i, s: (s[i], 0, 0)),
                pl.BlockSpec((1, 1, _H), lambda i, s: (s[i], 0, 0)),
            ],
            out_specs=pl.BlockSpec((_MT, _H), lambda i, s: (i, 0)),
            scratch_shapes=[pltpu.VMEM((_I, _H), jnp.bfloat16)],
        ),
        out_shape=jax.ShapeDtypeStruct((_NTM, _H), jnp.float32),
    )(tef, mid, W2, b2.reshape(_E, 1, _H))

    y = _combine(eout, d, w0r, w1r)

    return y.reshape(_B, _S, _H), aux.reshape(())


# speedup vs baseline: 1.0215x; 1.0146x over previous
import functools
import math

import jax
import jax.numpy as jnp
from jax import lax
from jax.experimental import pallas as pl
from jax.experimental.pallas import tpu as pltpu
from jax.experimental.pallas import tpu_sc as plsc

_B, _S, _H = 1, 2048, 1024
_E, _K, _I = 8, 2, 4096
_T = _B * _S

_MT = 128
_NT = 39
_NTM = _NT * _MT
_NTE = 40

_NW = 32
_TPW = _T // _NW
_CHT = 16


def _router_body(x_ref, wr_ref, br_ref,
                 d0_ref, d1_ref, w0_ref, w1_ref, te_ref, aux_ref):
    x = x_ref[...]
    logits = lax.dot_general(
        x.astype(jnp.bfloat16), wr_ref[...].astype(jnp.bfloat16),
        (((1,), (0,)), ((), ())),
        preferred_element_type=jnp.float32)
    logits = logits + br_ref[...]

    ii = lax.broadcasted_iota(jnp.int32, (_T, _E), 1)
    m1 = jnp.max(logits, axis=1, keepdims=True)
    idx1 = jnp.min(jnp.where(logits == m1, ii, _E), axis=1, keepdims=True)
    masked = jnp.where(ii == idx1, -jnp.inf, logits)
    m2 = jnp.max(masked, axis=1, keepdims=True)
    idx2 = jnp.min(jnp.where(masked == m2, ii, _E), axis=1, keepdims=True)

    e2 = jnp.exp(m2 - m1)
    ww1 = 1.0 / (1.0 + e2)
    ww2 = e2 * ww1
    oh1 = (ii == idx1)
    oh2 = (ii == idx2)
    oh = oh1.astype(jnp.float32) + oh2.astype(jnp.float32)
    cnt = jnp.sum(oh, axis=0, keepdims=True)

    z = jnp.exp(logits - m1)
    probs = z / jnp.sum(z, axis=1, keepdims=True)
    imp = jnp.sum(probs, axis=0, keepdims=True)
    impn = imp / jnp.sum(imp)
    loadn = cnt / jnp.sum(cnt)
    aux_ref[...] = (jnp.sum(impn * loadn) * float(_E)).reshape(1, 1)

    r = lax.broadcasted_iota(jnp.int32, (_T, _T), 0)
    c = lax.broadcasted_iota(jnp.int32, (_T, _T), 1)
    tri = (r > c).astype(jnp.bfloat16)
    excl = lax.dot_general(tri, oh.astype(jnp.bfloat16),
                           (((1,), (0,)), ((), ())),
                           preferred_element_type=jnp.float32)

    cap = jnp.ceil(cnt * (1.0 / _MT)) * float(_MT)
    r8 = lax.broadcasted_iota(jnp.int32, (_E, _E), 0)
    c8 = lax.broadcasted_iota(jnp.int32, (_E, _E), 1)
    u8 = (r8 < c8).astype(jnp.bfloat16)
    off = lax.dot_general(cap.astype(jnp.bfloat16), u8,
                          (((1,), (0,)), ((), ())),
                          preferred_element_type=jnp.float32)

    offb = jnp.broadcast_to(off, (_T, _E))
    d0 = jnp.sum(jnp.where(oh1, offb + excl, 0.0), axis=1, keepdims=True)
    d1 = jnp.sum(jnp.where(oh2, offb + excl, 0.0), axis=1, keepdims=True)
    d0_ref[...] = d0.astype(jnp.int32)
    d1_ref[...] = d1.astype(jnp.int32)
    w0_ref[...] = jnp.broadcast_to(ww1, (_T, 16))
    w1_ref[...] = jnp.broadcast_to(ww2, (_T, 16))

    ts = lax.broadcasted_iota(jnp.int32, (_NTE, _E), 0).astype(jnp.float32)
    ts = ts * float(_MT)
    ends = jnp.broadcast_to(off + cap, (_NTE, _E))
    te = jnp.sum((ts >= ends).astype(jnp.int32), axis=1, keepdims=True)
    te_ref[...] = jnp.minimum(te, _E - 1)


def _gelu_exact(v):
    return v * 0.5 * (1.0 + lax.erf(v * (1.0 / math.sqrt(2.0))))


def _seg_changed(te_ref):
    i = pl.program_id(0)
    prev = te_ref[jnp.maximum(i - 1, 0)]
    return (i == 0) | (te_ref[i] != prev)


def _ffn1_body(te_ref, xs_ref, w1_ref, b1_ref, mid_ref, w1b):
    @pl.when(_seg_changed(te_ref))
    def _():
        w1b[...] = w1_ref[0].astype(jnp.bfloat16)

    x = xs_ref[...].astype(jnp.bfloat16)
    mid = lax.dot_general(
        x, w1b[...], (((1,), (0,)), ((), ())),
        preferred_element_type=jnp.float32)
    mid_ref[...] = _gelu_exact(mid + b1_ref[0]).astype(jnp.bfloat16)


def _ffn2_body(te_ref, mid_ref, w2_ref, b2_ref, out_ref, w2b):
    @pl.when(_seg_changed(te_ref))
    def _():
        w2b[...] = w2_ref[0].astype(jnp.bfloat16)

    out_ref[...] = lax.dot_general(
        mid_ref[...], w2b[...], (((1,), (0,)), ((), ())),
        preferred_element_type=jnp.float32) + b2_ref[0]


@functools.cache
def _sc_kernels():
    mesh = plsc.VectorSubcoreMesh(core_axis_name="c", subcore_axis_name="s")

    @functools.partial(
        pl.kernel,
        out_type=jax.ShapeDtypeStruct((_NTM, _H), jnp.float32),
        mesh=mesh,
        scratch_types=[
            pltpu.VMEM((_TPW, _H), jnp.float32),
            pltpu.VMEM((2, _TPW), jnp.int32),
            pltpu.SemaphoreType.DMA,
        ],
    )
    def dispatch(x_hbm, d_hbm, xs_hbm, xv, idxv, sem):
        wid = lax.axis_index("s") * 2 + lax.axis_index("c")
        base = wid * _TPW
        pltpu.sync_copy(x_hbm.at[pl.ds(base, _TPW)], xv)
        pltpu.sync_copy(d_hbm.at[0, pl.ds(base, _TPW)], idxv.at[0])
        pltpu.sync_copy(d_hbm.at[1, pl.ds(base, _TPW)], idxv.at[1])
        c0 = pltpu.async_copy(xv, xs_hbm.at[idxv.at[0]], sem)
        c1 = pltpu.async_copy(xv, xs_hbm.at[idxv.at[1]], sem)
        c0.wait()
        c1.wait()

    @functools.partial(
        pl.kernel,
        out_type=jax.ShapeDtypeStruct((_T, _H), jnp.float32),
        mesh=mesh,
        scratch_types=[
            pltpu.VMEM((_CHT, _H), jnp.float32),
            pltpu.VMEM((_CHT, _H), jnp.float32),
            pltpu.VMEM((_CHT, _H), jnp.float32),
            pltpu.VMEM((_CHT, _H), jnp.float32),
            pltpu.VMEM((2, _TPW), jnp.int32),
            pltpu.VMEM((_TPW, 16), jnp.float32),
            pltpu.VMEM((_TPW, 16), jnp.float32),
            pltpu.SemaphoreType.DMA,
            pltpu.SemaphoreType.DMA,
        ],
    )
    def combine(eout_hbm, d_hbm, w0_hbm, w1_hbm, y_hbm,
                g0a, g1a, g0b, g1b, idxv, wv0, wv1, sema, semb):
        wid = lax.axis_index("s") * 2 + lax.axis_index("c")
        tb = wid * _TPW
        pltpu.sync_copy(d_hbm.at[0, pl.ds(tb, _TPW)], idxv.at[0])
        pltpu.sync_copy(d_hbm.at[1, pl.ds(tb, _TPW)], idxv.at[1])
        pltpu.sync_copy(w0_hbm.at[pl.ds(tb, _TPW)], wv0)
        pltpu.sync_copy(w1_hbm.at[pl.ds(tb, _TPW)], wv1)

        sets = ((g0a, g1a, sema), (g0b, g1b, semb))
        nch = _TPW // _CHT

        def fire(ci):
            g0, g1, sem = sets[ci % 2]
            s = pl.ds(ci * _CHT, _CHT)
            c0 = pltpu.async_copy(eout_hbm.at[idxv.at[0, s]], g0, sem)
            c1 = pltpu.async_copy(eout_hbm.at[idxv.at[1, s]], g1, sem)
            return c0, c1

        pending = fire(0)
        for ci in range(nch):
            nxt = fire(ci + 1) if ci + 1 < nch else None
            pending[0].wait()
            pending[1].wait()
            g0, g1, _ = sets[ci % 2]

            def tok_body(j, carry):
                a0 = wv0[ci * _CHT + j, :]
                a1 = wv1[ci * _CHT + j, :]

                def h_body(h, carry2):
                    for u in range(4):
                        s = pl.ds(h * 64 + u * 16, 16)
                        g0[j, s] = g0[j, s] * a0 + g1[j, s] * a1
                    return carry2

                return lax.fori_loop(0, _H // 64, h_body, carry)

            lax.fori_loop(0, _CHT, tok_body, 0)
            pltpu.sync_copy(g0, y_hbm.at[pl.ds(tb + ci * _CHT, _CHT)])
            pending = nxt

    return dispatch, combine


def kernel(hidden_states, Wr, br, W1, b1, W2, b2):
    x = hidden_states.reshape(_T, _H)

    d0, d1, w0r, w1r, te, aux = pl.pallas_call(
        _router_body,
        out_shape=(
            jax.ShapeDtypeStruct((_T, 1), jnp.int32),
            jax.ShapeDtypeStruct((_T, 1), jnp.int32),
            jax.ShapeDtypeStruct((_T, 16), jnp.float32),
            jax.ShapeDtypeStruct((_T, 16), jnp.float32),
            jax.ShapeDtypeStruct((_NTE, 1), jnp.int32),
            jax.ShapeDtypeStruct((1, 1), jnp.float32),
        ),
    )(x, Wr, br.reshape(1, _E))

    d = jnp.concatenate([d0.reshape(1, _T), d1.reshape(1, _T)], axis=0)

    _dispatch, _combine = _sc_kernels()
    xs = _dispatch(x, d)

    tef = te.reshape(_NTE)
    mid = pl.pallas_call(
        _ffn1_body,
        grid_spec=pltpu.PrefetchScalarGridSpec(
            num_scalar_prefetch=1,
            grid=(_NT,),
            in_specs=[
                pl.BlockSpec((_MT, _H), lambda i, s: (i, 0)),
                pl.BlockSpec((1, _H, _I), lambda i, s: (s[i], 0, 0)),
                pl.BlockSpec((1, 1, _I), lambda i, s: (s[i], 0, 0)),
            ],
            out_specs=pl.BlockSpec((_MT, _I), lambda i, s: (i, 0)),
            scratch_shapes=[pltpu.VMEM((_H, _I), jnp.bfloat16)],
        ),
        out_shape=jax.ShapeDtypeStruct((_NTM, _I), jnp.bfloat16),
    )(tef, xs, W1, b1.reshape(_E, 1, _I))

    eout = pl.pallas_call(
        _ffn2_body,
        grid_spec=pltpu.PrefetchScalarGridSpec(
            num_scalar_prefetch=1,
            grid=(_NT,),
            in_specs=[
                pl.BlockSpec((_MT, _I), lambda i, s: (i, 0)),
                pl.BlockSpec((1, _I, _H), lambda i, s: (s[i], 0, 0)),
                pl.BlockSpec((1, 1, _H), lambda i, s: (s[i], 0, 0)),
            ],
            out_specs=pl.BlockSpec((_MT, _H), lambda i, s: (i, 0)),
            scratch_shapes=[pltpu.VMEM((_I, _H), jnp.bfloat16)],
        ),
        out_shape=jax.ShapeDtypeStruct((_NTM, _H), jnp.float32),
    )(tef, mid, W2, b2.reshape(_E, 1, _H))

    y = _combine(eout, d, w0r, w1r)

    return y.reshape(_B, _S, _H), aux.reshape(())
